# baseline (speedup 1.0000x reference)
import functools

import jax
import jax.numpy as jnp
from jax import lax
from jax.experimental import pallas as pl
from jax.experimental.pallas import tpu as pltpu
from jax.experimental.pallas import tpu_sc as plsc

_INFO = plsc.get_sparse_core_info()
_NC = _INFO.num_cores
_NS = _INFO.num_subcores
_NW = _NC * _NS


@functools.lru_cache(maxsize=None)
def _make_gather(B, S, V, D):
    N = B * S
    assert N % _NW == 0
    b_per_w = N // _NW
    assert S % b_per_w == 0
    w_per_row = S // b_per_w
    chunk = min(b_per_w, 64)
    assert b_per_w % chunk == 0
    n_chunks = b_per_w // chunk

    mesh = plsc.VectorSubcoreMesh(core_axis_name="c", subcore_axis_name="s")

    @functools.partial(
        pl.kernel,
        mesh=mesh,
        out_type=jax.ShapeDtypeStruct((B, S, D), jnp.float32),
        scratch_types=[
            pltpu.VMEM((b_per_w,), jnp.int32),
            pltpu.VMEM((chunk, D), jnp.float32),
            pltpu.VMEM((chunk, D), jnp.float32),
            pltpu.SemaphoreType.DMA,
            pltpu.SemaphoreType.DMA,
            pltpu.SemaphoreType.DMA,
            pltpu.SemaphoreType.DMA,
        ],
    )
    def gather(table_hbm, idx_hbm, out_hbm, idx_v, rows0, rows1,
               isem0, isem1, osem0, osem1):
        wid = lax.axis_index("s") * _NC + lax.axis_index("c")
        row = wid // w_per_row
        col = (wid % w_per_row) * b_per_w
        rows = (rows0, rows1)
        isems = (isem0, isem1)
        osems = (osem0, osem1)
        pltpu.sync_copy(idx_hbm.at[row, pl.ds(col, b_per_w)], idx_v)

        def idx_slice(g):
            return idx_v.at[pl.ds(g * chunk, chunk)]

        in_cp = [None] * n_chunks
        out_cp = [None] * n_chunks
        in_cp[0] = pltpu.async_copy(table_hbm.at[idx_slice(0)], rows[0],
                                    isems[0])
        for g in range(n_chunks):
            b = g % 2
            if g + 1 < n_chunks:
                if g - 1 >= 0:
                    out_cp[g - 1].wait()
                in_cp[g + 1] = pltpu.async_copy(
                    table_hbm.at[idx_slice(g + 1)], rows[1 - b],
                    isems[1 - b])
            in_cp[g].wait()
            out_cp[g] = pltpu.async_copy(
                rows[b], out_hbm.at[row, pl.ds(col + g * chunk, chunk)],
                osems[b])
        for g in range(max(0, n_chunks - 2), n_chunks):
            out_cp[g].wait()

    return gather


def kernel(sequence, token_table):
    B, S = sequence.shape
    V, D = token_table.shape
    idx = sequence.astype(jnp.int32)
    return _make_gather(B, S, V, D)(token_table, idx)


# speedup vs baseline: 1.0283x; 1.0283x over previous
import functools

import jax
import jax.numpy as jnp
from jax import lax
from jax.experimental import pallas as pl
from jax.experimental.pallas import tpu as pltpu
from jax.experimental.pallas import tpu_sc as plsc

_INFO = plsc.get_sparse_core_info()
_NC = _INFO.num_cores
_NS = _INFO.num_subcores
_NW = _NC * _NS


@functools.lru_cache(maxsize=None)
def _make_gather(B, S, V, D):
    N = B * S
    assert N % _NW == 0
    b_per_w = N // _NW
    assert S % b_per_w == 0
    w_per_row = S // b_per_w
    chunk = min(b_per_w, 32)
    assert b_per_w % chunk == 0
    n_chunks = b_per_w // chunk
    nbuf = min(n_chunks, 4)

    mesh = plsc.VectorSubcoreMesh(core_axis_name="c", subcore_axis_name="s")

    @functools.partial(
        pl.kernel,
        mesh=mesh,
        out_type=jax.ShapeDtypeStruct((B, S, D), jnp.float32),
        scratch_types=(
            [pltpu.VMEM((b_per_w,), jnp.int32)]
            + [pltpu.VMEM((chunk, D), jnp.float32) for _ in range(nbuf)]
            + [pltpu.SemaphoreType.DMA for _ in range(2 * nbuf)]
        ),
    )
    def gather(table_hbm, idx_hbm, out_hbm, idx_v, *bufs_and_sems):
        rows = bufs_and_sems[:nbuf]
        isems = bufs_and_sems[nbuf:2 * nbuf]
        osems = bufs_and_sems[2 * nbuf:]
        wid = lax.axis_index("s") * _NC + lax.axis_index("c")
        row = wid // w_per_row
        col = (wid % w_per_row) * b_per_w
        pltpu.sync_copy(idx_hbm.at[row, pl.ds(col, b_per_w)], idx_v)

        def start_gather(g):
            b = g % nbuf
            return pltpu.async_copy(
                table_hbm.at[idx_v.at[pl.ds(g * chunk, chunk)]],
                rows[b], isems[b])

        in_cp = [None] * n_chunks
        out_cp = [None] * n_chunks
        for g in range(min(nbuf - 1, n_chunks)):
            in_cp[g] = start_gather(g)
        for g in range(n_chunks):
            b = g % nbuf
            ng = g + nbuf - 1
            if ng < n_chunks:
                if g >= 1:
                    out_cp[g - 1].wait()
                in_cp[ng] = start_gather(ng)
            in_cp[g].wait()
            out_cp[g] = pltpu.async_copy(
                rows[b], out_hbm.at[row, pl.ds(col + g * chunk, chunk)],
                osems[b])
        for g in range(max(0, n_chunks - nbuf + 1), n_chunks):
            out_cp[g].wait()

    return gather


def kernel(sequence, token_table):
    B, S = sequence.shape
    V, D = token_table.shape
    idx = sequence.astype(jnp.int32)
    return _make_gather(B, S, V, D)(token_table, idx)
